# baseline (speedup 1.0000x reference)
import functools

import jax
import jax.numpy as jnp
from jax import lax
from jax.experimental import pallas as pl
from jax.experimental.pallas import tpu as pltpu
from jax.experimental.pallas import tpu_sc as plsc

_ROW = 768
_CH = 32
_L = 16


def _build_sc_call(B, NC, NS, num_kp, d_dist, d_state):
    NW = NC * NS
    b_per_w = B // NW
    n_ch = b_per_w // _CH
    d_out = d_dist + d_state
    mesh = plsc.VectorSubcoreMesh(core_axis_name="c", subcore_axis_name="s")

    @functools.partial(
        pl.kernel,
        mesh=mesh,
        out_type=(
            jax.ShapeDtypeStruct((B, _ROW), jnp.float32),
            jax.ShapeDtypeStruct((NW * 2, _ROW), jnp.float32),
        ),
        scratch_types=[
            pltpu.VMEM((num_kp * d_dist,), jnp.float32),
            pltpu.VMEM((2 * d_state,), jnp.float32),
            pltpu.VMEM((2, _ROW), jnp.float32),
            pltpu.VMEM((b_per_w,), jnp.int32),
            pltpu.VMEM((n_ch, _CH), jnp.int32),
            pltpu.VMEM((4, _CH, _ROW), jnp.float32),
        ] + [pltpu.SemaphoreType.DMA] * 8,
    )
    def sc_gather(dist_hbm, state_hbm, idx_hbm, out_hbm, table_hbm,
                  dist_v, state_v, fused_v, idx_v, idx2_v, rows_v, *sems):
        wid = lax.axis_index("s") * NC + lax.axis_index("c")
        base = wid * b_per_w

        pltpu.sync_copy(dist_hbm, dist_v)
        pltpu.sync_copy(state_hbm, state_v)
        pltpu.sync_copy(idx_hbm.at[pl.ds(base, b_per_w)], idx_v)

        for g in range(2):
            for k in range(num_kp):
                col = k * d_out
                for j in range(d_dist // _L):
                    fused_v[g, pl.ds(col + j * _L, _L)] = (
                        dist_v[pl.ds(k * d_dist + j * _L, _L)])
                for j in range(d_state // _L):
                    fused_v[g, pl.ds(col + d_dist + j * _L, _L)] = (
                        state_v[pl.ds(g * d_state + j * _L, _L)])
        pltpu.sync_copy(fused_v, table_hbm.at[pl.ds(2 * wid, 2)])

        off = jnp.broadcast_to(2 * wid, (_L,)).astype(jnp.int32)
        for c in range(n_ch):
            for j in range(_CH // _L):
                idx2_v[c, pl.ds(j * _L, _L)] = (
                    idx_v[pl.ds(c * _CH + j * _L, _L)] + off)

        NB = 4
        gsem = sems[:NB]
        ssem = sems[NB:]
        gat = [None] * NB
        sca = [None] * NB
        for c in range(min(NB, n_ch)):
            gat[c] = pltpu.async_copy(
                table_hbm.at[idx2_v.at[c]], rows_v.at[c], gsem[c])
        for c in range(n_ch):
            p = c % NB
            gat[p].wait()
            sca[p] = pltpu.async_copy(
                rows_v.at[p], out_hbm.at[pl.ds(base + c * _CH, _CH)], ssem[p])
            if c + NB < n_ch:
                sca[p].wait()
                gat[p] = pltpu.async_copy(
                    table_hbm.at[idx2_v.at[c + NB]], rows_v.at[p], gsem[p])
                sca[p] = None
        for p in range(NB):
            if sca[p] is not None:
                sca[p].wait()

    return sc_gather


def kernel(grip_state, distinction_table, state_table):
    B = grip_state.shape[0]
    num_kp, d_dist = distinction_table.shape
    d_state = state_table.shape[-1]
    info = plsc.get_sparse_core_info()
    NC, NS = info.num_cores, info.num_subcores

    out, _ = _build_sc_call(B, NC, NS, num_kp, d_dist, d_state)(
        distinction_table.reshape(-1),
        state_table.reshape(-1),
        grip_state.astype(jnp.int32))
    return out.reshape(B, num_kp, d_dist + d_state)
